# baseline (speedup 1.0000x reference)
import functools

import jax
import jax.numpy as jnp
from jax.experimental import pallas as pl
from jax.experimental.pallas import tpu as pltpu

_F32 = jnp.float32
_BF16 = jnp.bfloat16


def _cspec(shape):
    return pl.BlockSpec(shape, lambda *_, _s=shape: (0,) * len(_s))


def _expand(x2d, ew, eb):
    e = jnp.dot(x2d, ew[...], preferred_element_type=_F32)
    return jnp.clip(e + eb[...], 0.0, 6.0)


def _proj(acc2d, pw, pb):
    return jnp.dot(acc2d.astype(_BF16), pw[...], preferred_element_type=_F32) + pb[...]


def _fill_scratch(scr, e3, h, w, hid):
    g = hid // 128
    for gi in range(g):
        scr[gi, 1:h + 1, 1:w + 1, :] = e3[..., 128 * gi:128 * (gi + 1)]
        scr[gi, 0:1, 0:w + 2, :] = jnp.zeros((1, w + 2, 128), _F32)
        scr[gi, h + 1:h + 2, 0:w + 2, :] = jnp.zeros((1, w + 2, 128), _F32)
        scr[gi, 0:h + 2, 0:1, :] = jnp.zeros((h + 2, 1, 128), _F32)
        scr[gi, 0:h + 2, w + 1:w + 2, :] = jnp.zeros((h + 2, 1, 128), _F32)
    return g


def _dw_taps(scr, dw, g, ho, wo, stride):
    accs = []
    for gi in range(g):
        a = jnp.zeros((ho, wo, 128), _F32)
        for kh in range(3):
            for kw in range(3):
                if stride == 1:
                    tap = scr[gi, kh:kh + ho, kw:kw + wo, :]
                else:
                    tap = scr[gi, pl.ds(kh, ho, 2), pl.ds(kw, wo, 2), :]
                a = a + tap * dw[kh, kw, 128 * gi:128 * (gi + 1)]
        accs.append(a)
    return jnp.concatenate(accs, axis=-1) if g > 1 else accs[0]


def _cb_s1(x, scr, ew, eb, dw, db, pw, pb, use_res):
    h, w, c = x.shape
    hid = ew.shape[1]
    m = h * w
    e = _expand(x.reshape(m, c), ew, eb).reshape(h, w, hid)
    g = _fill_scratch(scr, e, h, w, hid)
    acc = _dw_taps(scr, dw, g, h, w, 1)
    acc = jnp.clip(acc + db[...], 0.0, 6.0)
    y = _proj(acc.reshape(m, hid), pw, pb)
    if use_res:
        y = y + x.reshape(m, c).astype(_F32)
    return y.astype(_BF16).reshape(h, w, pw.shape[1])


def _cb_s2(x, scr, ew, eb, dw, db, pw, pb):
    h, w, c = x.shape
    ho, wo = h // 2, w // 2
    hid = ew.shape[1]
    e = _expand(x.reshape(h * w, c), ew, eb).reshape(h, w, hid)
    g = _fill_scratch(scr, e, h, w, hid)
    acc = _dw_taps(scr, dw, g, ho, wo, 2)
    acc = jnp.clip(acc + db[...], 0.0, 6.0)
    y = _proj(acc.reshape(ho * wo, hid), pw, pb)
    return y.astype(_BF16).reshape(ho, wo, pw.shape[1])


def _stem_f1_body(xe_ref, xo_ref, w_ref, b_ref, dw, db, pw, pb, o_ref, scrq, scrh):
    _, _, ho, w2 = xe_ref.shape
    wo = w2
    hid = dw.shape[2]
    hi = jnp.uint32(0xFFFF0000)
    ve = pltpu.bitcast(xe_ref[0], jnp.uint32)
    vo = pltpu.bitcast(xo_ref[0], jnp.uint32)
    for par, v in ((0, ve), (1, vo)):
        p0 = pltpu.bitcast(v << 16, _F32)
        p1 = pltpu.bitcast(v & hi, _F32)
        p1s = jnp.concatenate(
            [jnp.zeros((3, ho, 1), _F32), p1[:, :, :w2 - 1]], axis=2)
        off = par
        scrq[0 + par, :, off:ho + off, 0:w2] = p0
        scrq[2 + par, :, off:ho + off, 0:w2] = p1
        scrq[4 + par, :, off:ho + off, 0:w2] = p1s
    scrq[1, :, 0:1, :] = jnp.zeros((3, 1, 128), _F32)
    scrq[3, :, 0:1, :] = jnp.zeros((3, 1, 128), _F32)
    scrq[5, :, 0:1, :] = jnp.zeros((3, 1, 128), _F32)
    wp_of_kw = {0: 4, 1: 0, 2: 2}
    taps = []
    for kh in range(3):
        par = 1 if kh != 1 else 0
        lo = 1 if kh == 2 else 0
        for kw in range(3):
            for c in range(3):
                taps.append(
                    scrq[wp_of_kw[kw] + par, c, lo:lo + ho, :].astype(_BF16))
    zrow = jnp.zeros((ho, 128), _BF16)
    p3 = jnp.stack(taps + [zrow] * 5, axis=1)
    y = jnp.einsum('atb,tn->abn', p3, w_ref[...],
                   preferred_element_type=_F32) + b_ref[...]
    y = jnp.clip(y, 0.0, 6.0)
    scrh[1:ho + 1, 1:wo + 1, :] = y[:, :wo, :].astype(_BF16).astype(_F32)
    scrh[0:1, :, :] = jnp.zeros((1, wo + 2, hid), _F32)
    scrh[ho + 1:ho + 2, :, :] = jnp.zeros((1, wo + 2, hid), _F32)
    scrh[:, 0:1, :] = jnp.zeros((ho + 2, 1, hid), _F32)
    scrh[:, wo + 1:wo + 2, :] = jnp.zeros((ho + 2, 1, hid), _F32)
    acc = jnp.zeros((ho, wo, hid), _F32)
    for kh in range(3):
        for kw in range(3):
            acc = acc + scrh[kh:kh + ho, kw:kw + wo, :] * dw[kh, kw, :]
    acc = jnp.clip(acc + db[...], 0.0, 6.0)
    y1 = _proj(acc.reshape(ho * wo, hid), pw, pb)
    o_ref[0] = y1.astype(_BF16).reshape(ho, wo, pw.shape[1])


def _stem_f1(xe, xo, w, b, dw, db, pw, pb):
    n, _, ho, w2 = xe.shape
    wo = w2
    hid = dw.shape[2]
    cout = pw.shape[1]
    w32 = jnp.pad(w, ((0, 32 - w.shape[0]), (0, 0)))
    return pl.pallas_call(
        _stem_f1_body,
        grid=(n,),
        in_specs=[pl.BlockSpec((1, 3, ho, w2), lambda i: (i, 0, 0, 0)),
                  pl.BlockSpec((1, 3, ho, w2), lambda i: (i, 0, 0, 0)),
                  _cspec((32, 128)), _cspec((1, 128)),
                  _cspec((3, 3, hid)), _cspec((1, hid)),
                  _cspec((hid, cout)), _cspec((1, cout))],
        out_specs=pl.BlockSpec((1, ho, wo, cout), lambda i: (i, 0, 0, 0)),
        out_shape=jax.ShapeDtypeStruct((n, ho, wo, cout), _BF16),
        scratch_shapes=[pltpu.VMEM((6, 3, ho + 1, 128), _F32),
                        pltpu.VMEM((ho + 2, wo + 2, hid), _F32)],
        compiler_params=pltpu.CompilerParams(
            dimension_semantics=("parallel",),
            vmem_limit_bytes=64 * 1024 * 1024),
    )(xe, xo, w32, b.reshape(1, 128), dw, db.reshape(1, hid),
      pw, pb.reshape(1, cout))


def _make_tail_body(specs):
    nb = len(specs)

    def body(*refs):
        x_ref = refs[0]
        w18, b18 = refs[1 + 6 * nb], refs[2 + 6 * nb]
        o17_ref, opool_ref = refs[3 + 6 * nb], refs[4 + 6 * nb]
        scr_big, scr_small = refs[5 + 6 * nb], refs[6 + 6 * nb]
        x = x_ref[0]
        for bi, (stride, use_res, use_big) in enumerate(specs):
            scr = scr_big if use_big else scr_small
            ew, eb, dw, db, pw, pb = refs[1 + 6 * bi:7 + 6 * bi]
            if stride == 1:
                x = _cb_s1(x, scr, ew, eb, dw, db, pw, pb, use_res)
            else:
                x = _cb_s2(x, scr, ew, eb, dw, db, pw, pb)
        o17_ref[0] = x
        h, w, c = x.shape
        z = jnp.dot(x.reshape(h * w, c), w18[...], preferred_element_type=_F32)
        z = jnp.clip(z + b18[...], 0.0, 6.0).astype(_BF16)
        pooled = jnp.mean(z.astype(_F32), axis=0, keepdims=True)
        opool_ref[0] = pooled.astype(_BF16)

    return body


def _tail_chain(x, blocks, w18, b18):
    n, h, w, c = x.shape
    args, in_specs = [x], [pl.BlockSpec((1, h, w, c), lambda i: (i, 0, 0, 0))]
    specs = []
    big, small = [1, 4], [1, 4]
    ch, cw, cc = h, w, c
    for (ew, eb, dw, db, pw, pb, stride, use_res) in blocks:
        hid = ew.shape[1]
        cout = pw.shape[1]
        use_big = ch > 16
        tgt = big if use_big else small
        tgt[0] = max(tgt[0], hid // 128)
        tgt[1] = max(tgt[1], ch + 2)
        specs.append((stride, use_res, use_big))
        in_specs += [_cspec((cc, hid)), _cspec((1, hid)), _cspec((3, 3, hid)),
                     _cspec((1, hid)), _cspec((hid, cout)), _cspec((1, cout))]
        args += [ew, eb.reshape(1, hid), dw, db.reshape(1, hid),
                 pw, pb.reshape(1, cout)]
        if stride == 2:
            ch, cw = ch // 2, cw // 2
        cc = cout
    n1280 = w18.shape[1]
    in_specs += [_cspec((cc, n1280)), _cspec((1, n1280))]
    args += [w18, b18.reshape(1, n1280)]
    o17, pooled = pl.pallas_call(
        _make_tail_body(specs),
        grid=(n,),
        in_specs=in_specs,
        out_specs=[pl.BlockSpec((1, ch, cw, cc), lambda i: (i, 0, 0, 0)),
                   pl.BlockSpec((1, 1, n1280), lambda i: (i, 0, 0))],
        out_shape=[jax.ShapeDtypeStruct((n, ch, cw, cc), _BF16),
                   jax.ShapeDtypeStruct((n, 1, n1280), _BF16)],
        scratch_shapes=[pltpu.VMEM((big[0], big[1], big[1], 128), _F32),
                        pltpu.VMEM((small[0], small[1], small[1], 128), _F32)],
        compiler_params=pltpu.CompilerParams(
            dimension_semantics=("parallel",),
            vmem_limit_bytes=64 * 1024 * 1024),
    )(*args)
    return o17, pooled


def _fc_body(p_ref, w_ref, b_ref, o_ref):
    p = p_ref[...]
    p2 = p.reshape(p.shape[0], p.shape[2])
    o_ref[...] = jnp.dot(p2, w_ref[...], preferred_element_type=_F32) + b_ref[...]


def _fc(pooled, w, b):
    n = pooled.shape[0]
    k = pooled.shape[2]
    fp = w.shape[1]
    return pl.pallas_call(
        _fc_body,
        grid=(1,),
        in_specs=[_cspec((n, 1, k)), _cspec((k, fp)), _cspec((1, fp))],
        out_specs=pl.BlockSpec((n, fp), lambda i: (0, 0)),
        out_shape=jax.ShapeDtypeStruct((n, fp), _F32),
    )(pooled, w, b.reshape(1, fp))


def kernel(x, f0_w, f0_b, f1_dw_w, f1_dw_b, f1_proj_w, f1_proj_b, f2_expand_w, f2_expand_b, f2_dw_w, f2_dw_b, f2_proj_w, f2_proj_b, f3_expand_w, f3_expand_b, f3_dw_w, f3_dw_b, f3_proj_w, f3_proj_b, f4_expand_w, f4_expand_b, f4_dw_w, f4_dw_b, f4_proj_w, f4_proj_b, f5_expand_w, f5_expand_b, f5_dw_w, f5_dw_b, f5_proj_w, f5_proj_b, f6_expand_w, f6_expand_b, f6_dw_w, f6_dw_b, f6_proj_w, f6_proj_b, f7_expand_w, f7_expand_b, f7_dw_w, f7_dw_b, f7_proj_w, f7_proj_b, f8_expand_w, f8_expand_b, f8_dw_w, f8_dw_b, f8_proj_w, f8_proj_b, f9_expand_w, f9_expand_b, f9_dw_w, f9_dw_b, f9_proj_w, f9_proj_b, f10_expand_w, f10_expand_b, f10_dw_w, f10_dw_b, f10_proj_w, f10_proj_b, f11_expand_w, f11_expand_b, f11_dw_w, f11_dw_b, f11_proj_w, f11_proj_b, f12_expand_w, f12_expand_b, f12_dw_w, f12_dw_b, f12_proj_w, f12_proj_b, f13_expand_w, f13_expand_b, f13_dw_w, f13_dw_b, f13_proj_w, f13_proj_b, f14_expand_w, f14_expand_b, f14_dw_w, f14_dw_b, f14_proj_w, f14_proj_b, f15_expand_w, f15_expand_b, f15_dw_w, f15_dw_b, f15_proj_w, f15_proj_b, f16_expand_w, f16_expand_b, f16_dw_w, f16_dw_b, f16_proj_w, f16_proj_b, f17_expand_w, f17_expand_b, f17_dw_w, f17_dw_b, f17_proj_w, f17_proj_b, f18_w, f18_b, fc_w, fc_b):
    n = x.shape[0]
    xbf = x.astype(_BF16)
    xi = jax.lax.bitcast_convert_type(
        xbf.reshape(n, 3, 224, 112, 2), jnp.int32)
    y1 = _stem_f1(xi[:, :, 0::2], xi[:, :, 1::2], f0_w, f0_b,
                  f1_dw_w, f1_dw_b, f1_proj_w, f1_proj_b)
    o17, pooled = _tail_chain(y1, [
        (f2_expand_w, f2_expand_b, f2_dw_w, f2_dw_b, f2_proj_w, f2_proj_b, 2, False),
        (f3_expand_w, f3_expand_b, f3_dw_w, f3_dw_b, f3_proj_w, f3_proj_b, 1, True),
        (f4_expand_w, f4_expand_b, f4_dw_w, f4_dw_b, f4_proj_w, f4_proj_b, 2, False),
        (f5_expand_w, f5_expand_b, f5_dw_w, f5_dw_b, f5_proj_w, f5_proj_b, 1, True),
        (f6_expand_w, f6_expand_b, f6_dw_w, f6_dw_b, f6_proj_w, f6_proj_b, 1, True),
        (f7_expand_w, f7_expand_b, f7_dw_w, f7_dw_b, f7_proj_w, f7_proj_b, 2, False),
        (f8_expand_w, f8_expand_b, f8_dw_w, f8_dw_b, f8_proj_w, f8_proj_b, 1, True),
        (f9_expand_w, f9_expand_b, f9_dw_w, f9_dw_b, f9_proj_w, f9_proj_b, 1, True),
        (f10_expand_w, f10_expand_b, f10_dw_w, f10_dw_b, f10_proj_w, f10_proj_b, 1, True),
        (f11_expand_w, f11_expand_b, f11_dw_w, f11_dw_b, f11_proj_w, f11_proj_b, 1, False),
        (f12_expand_w, f12_expand_b, f12_dw_w, f12_dw_b, f12_proj_w, f12_proj_b, 1, True),
        (f13_expand_w, f13_expand_b, f13_dw_w, f13_dw_b, f13_proj_w, f13_proj_b, 1, True),
        (f14_expand_w, f14_expand_b, f14_dw_w, f14_dw_b, f14_proj_w, f14_proj_b, 2, False),
        (f15_expand_w, f15_expand_b, f15_dw_w, f15_dw_b, f15_proj_w, f15_proj_b, 1, True),
        (f16_expand_w, f16_expand_b, f16_dw_w, f16_dw_b, f16_proj_w, f16_proj_b, 1, True),
        (f17_expand_w, f17_expand_b, f17_dw_w, f17_dw_b, f17_proj_w, f17_proj_b, 1, False),
    ], f18_w, f18_b)

    predict = _fc(pooled, fc_w, fc_b)[:, :12]

    feat = jnp.transpose(o17[..., :320].astype(_F32), (0, 3, 1, 2))
    feature = jnp.stack([feat[:n // 2], feat[n // 2:]])
    return feature, predict


# speedup vs baseline: 1.0788x; 1.0788x over previous
import jax
import jax.numpy as jnp
from jax.experimental import pallas as pl
from jax.experimental.pallas import tpu as pltpu

_F32 = jnp.float32
_BF16 = jnp.bfloat16


def _cspec(shape):
    return pl.BlockSpec(shape, lambda *_, _s=shape: (0,) * len(_s))


def _expand(x2d, ew, eb):
    e = jnp.dot(x2d, ew[...], preferred_element_type=_F32)
    return jnp.clip(e + eb[...], 0.0, 6.0)


def _proj(acc2d, pw, pb):
    return jnp.dot(acc2d.astype(_BF16), pw[...], preferred_element_type=_F32) + pb[...]


def _fill_scratch(scr, e3, h, w, hid):
    g = hid // 128
    for gi in range(g):
        scr[gi, 1:h + 1, 1:w + 1, :] = e3[..., 128 * gi:128 * (gi + 1)]
        scr[gi, 0:1, 0:w + 2, :] = jnp.zeros((1, w + 2, 128), _F32)
        scr[gi, h + 1:h + 2, 0:w + 2, :] = jnp.zeros((1, w + 2, 128), _F32)
        scr[gi, 0:h + 2, 0:1, :] = jnp.zeros((h + 2, 1, 128), _F32)
        scr[gi, 0:h + 2, w + 1:w + 2, :] = jnp.zeros((h + 2, 1, 128), _F32)
    return g


def _dw_taps(scr, dw, g, ho, wo, stride):
    accs = []
    for gi in range(g):
        a = jnp.zeros((ho, wo, 128), _F32)
        for kh in range(3):
            for kw in range(3):
                if stride == 1:
                    tap = scr[gi, kh:kh + ho, kw:kw + wo, :]
                else:
                    tap = scr[gi, pl.ds(kh, ho, 2), pl.ds(kw, wo, 2), :]
                a = a + tap * dw[kh, kw, 128 * gi:128 * (gi + 1)]
        accs.append(a)
    return jnp.concatenate(accs, axis=-1) if g > 1 else accs[0]


def _cb_s1(x, scr, ew, eb, dw, db, pw, pb, use_res):
    h, w, c = x.shape
    hid = ew.shape[1]
    m = h * w
    e = _expand(x.reshape(m, c), ew, eb).reshape(h, w, hid)
    g = _fill_scratch(scr, e, h, w, hid)
    acc = _dw_taps(scr, dw, g, h, w, 1)
    acc = jnp.clip(acc + db[...], 0.0, 6.0)
    y = _proj(acc.reshape(m, hid), pw, pb)
    if use_res:
        y = y + x.reshape(m, c).astype(_F32)
    return y.astype(_BF16).reshape(h, w, pw.shape[1])


def _cb_s2(x, scr, ew, eb, dw, db, pw, pb):
    h, w, c = x.shape
    ho, wo = h // 2, w // 2
    hid = ew.shape[1]
    e = _expand(x.reshape(h * w, c), ew, eb).reshape(h, w, hid)
    g = _fill_scratch(scr, e, h, w, hid)
    acc = _dw_taps(scr, dw, g, ho, wo, 2)
    acc = jnp.clip(acc + db[...], 0.0, 6.0)
    y = _proj(acc.reshape(ho * wo, hid), pw, pb)
    return y.astype(_BF16).reshape(ho, wo, pw.shape[1])


def _stem_f1_body(xi_ref, w_ref, b_ref, dw, db, pw, pb, o_ref, scrq, scrh):
    _, _, h, w2 = xi_ref.shape
    ho, wo = h // 2, w2
    hid = dw.shape[2]
    vu = pltpu.bitcast(xi_ref[0], jnp.uint32)
    ph0 = pltpu.bitcast(vu << 16, _F32)
    ph1 = pltpu.bitcast(vu & jnp.uint32(0xFFFF0000), _F32)
    ph1s = jnp.concatenate(
        [jnp.zeros((3, h, 1), _F32), ph1[:, :, :w2 - 1]], axis=2)
    scrq[0, :, 1:h + 1, 0:w2] = ph0
    scrq[1, :, 1:h + 1, 0:w2] = ph1
    scrq[2, :, 1:h + 1, 0:w2] = ph1s
    scrq[:, :, 0:1, :] = jnp.zeros((3, 3, 1, 128), _F32)
    scrq[:, :, h + 1:h + 2, :] = jnp.zeros((3, 3, 1, 128), _F32)
    plane_of_kw = {0: 2, 1: 0, 2: 1}
    taps = [scrq[plane_of_kw[kw], c, pl.ds(kh, ho, 2), :].astype(_BF16)
            for kh in range(3) for kw in range(3) for c in range(3)]
    zrow = jnp.zeros((ho, 128), _BF16)
    p3 = jnp.stack(taps + [zrow] * 5, axis=1)
    y = jnp.einsum('atb,tn->abn', p3, w_ref[...],
                   preferred_element_type=_F32) + b_ref[...]
    y = jnp.clip(y, 0.0, 6.0)
    scrh[1:ho + 1, 1:wo + 1, :] = y[:, :wo, :].astype(_BF16).astype(_F32)
    scrh[0:1, :, :] = jnp.zeros((1, wo + 2, hid), _F32)
    scrh[ho + 1:ho + 2, :, :] = jnp.zeros((1, wo + 2, hid), _F32)
    scrh[:, 0:1, :] = jnp.zeros((ho + 2, 1, hid), _F32)
    scrh[:, wo + 1:wo + 2, :] = jnp.zeros((ho + 2, 1, hid), _F32)
    acc = jnp.zeros((ho, wo, hid), _F32)
    for kh in range(3):
        for kw in range(3):
            acc = acc + scrh[kh:kh + ho, kw:kw + wo, :] * dw[kh, kw, :]
    acc = jnp.clip(acc + db[...], 0.0, 6.0)
    y1 = _proj(acc.reshape(ho * wo, hid), pw, pb)
    o_ref[0] = y1.astype(_BF16).reshape(ho, wo, pw.shape[1])


def _stem_f1(xi, w, b, dw, db, pw, pb):
    n, _, h, w2 = xi.shape
    ho, wo = h // 2, w2
    hid = dw.shape[2]
    cout = pw.shape[1]
    w32 = jnp.pad(w, ((0, 32 - w.shape[0]), (0, 0)))
    return pl.pallas_call(
        _stem_f1_body,
        grid=(n,),
        in_specs=[pl.BlockSpec((1, 3, h, w2), lambda i: (i, 0, 0, 0)),
                  _cspec((32, 128)), _cspec((1, 128)),
                  _cspec((3, 3, hid)), _cspec((1, hid)),
                  _cspec((hid, cout)), _cspec((1, cout))],
        out_specs=pl.BlockSpec((1, ho, wo, cout), lambda i: (i, 0, 0, 0)),
        out_shape=jax.ShapeDtypeStruct((n, ho, wo, cout), _BF16),
        scratch_shapes=[pltpu.VMEM((3, 3, h + 2, 128), _F32),
                        pltpu.VMEM((ho + 2, wo + 2, hid), _F32)],
        compiler_params=pltpu.CompilerParams(
            dimension_semantics=("parallel",),
            vmem_limit_bytes=64 * 1024 * 1024),
    )(xi, w32, b.reshape(1, 128), dw, db.reshape(1, hid), pw, pb.reshape(1, cout))


def _make_tail_body(specs):
    nb = len(specs)

    def body(*refs):
        x_ref = refs[0]
        w18, b18 = refs[1 + 6 * nb], refs[2 + 6 * nb]
        o17_ref, opool_ref = refs[3 + 6 * nb], refs[4 + 6 * nb]
        scr_big, scr_small = refs[5 + 6 * nb], refs[6 + 6 * nb]
        x = x_ref[0]
        for bi, (stride, use_res, use_big) in enumerate(specs):
            scr = scr_big if use_big else scr_small
            ew, eb, dw, db, pw, pb = refs[1 + 6 * bi:7 + 6 * bi]
            if stride == 1:
                x = _cb_s1(x, scr, ew, eb, dw, db, pw, pb, use_res)
            else:
                x = _cb_s2(x, scr, ew, eb, dw, db, pw, pb)
        o17_ref[0] = x
        h, w, c = x.shape
        z = jnp.dot(x.reshape(h * w, c), w18[...], preferred_element_type=_F32)
        z = jnp.clip(z + b18[...], 0.0, 6.0).astype(_BF16)
        pooled = jnp.mean(z.astype(_F32), axis=0, keepdims=True)
        opool_ref[0] = pooled.astype(_BF16)

    return body


def _tail_chain(x, blocks, w18, b18):
    n, h, w, c = x.shape
    args, in_specs = [x], [pl.BlockSpec((1, h, w, c), lambda i: (i, 0, 0, 0))]
    specs = []
    big, small = [1, 4], [1, 4]
    ch, cw, cc = h, w, c
    for (ew, eb, dw, db, pw, pb, stride, use_res) in blocks:
        hid = ew.shape[1]
        cout = pw.shape[1]
        use_big = ch > 16
        tgt = big if use_big else small
        tgt[0] = max(tgt[0], hid // 128)
        tgt[1] = max(tgt[1], ch + 2)
        specs.append((stride, use_res, use_big))
        in_specs += [_cspec((cc, hid)), _cspec((1, hid)), _cspec((3, 3, hid)),
                     _cspec((1, hid)), _cspec((hid, cout)), _cspec((1, cout))]
        args += [ew, eb.reshape(1, hid), dw, db.reshape(1, hid),
                 pw, pb.reshape(1, cout)]
        if stride == 2:
            ch, cw = ch // 2, cw // 2
        cc = cout
    n1280 = w18.shape[1]
    in_specs += [_cspec((cc, n1280)), _cspec((1, n1280))]
    args += [w18, b18.reshape(1, n1280)]
    o17, pooled = pl.pallas_call(
        _make_tail_body(specs),
        grid=(n,),
        in_specs=in_specs,
        out_specs=[pl.BlockSpec((1, ch, cw, cc), lambda i: (i, 0, 0, 0)),
                   pl.BlockSpec((1, 1, n1280), lambda i: (i, 0, 0))],
        out_shape=[jax.ShapeDtypeStruct((n, ch, cw, cc), _BF16),
                   jax.ShapeDtypeStruct((n, 1, n1280), _BF16)],
        scratch_shapes=[pltpu.VMEM((big[0], big[1], big[1], 128), _F32),
                        pltpu.VMEM((small[0], small[1], small[1], 128), _F32)],
        compiler_params=pltpu.CompilerParams(
            dimension_semantics=("parallel",),
            vmem_limit_bytes=64 * 1024 * 1024),
    )(*args)
    return o17, pooled


def _fc_body(p_ref, w_ref, b_ref, o_ref):
    p = p_ref[...]
    p2 = p.reshape(p.shape[0], p.shape[2])
    o_ref[...] = jnp.dot(p2, w_ref[...], preferred_element_type=_F32) + b_ref[...]


def _fc(pooled, w, b):
    n = pooled.shape[0]
    k = pooled.shape[2]
    fp = w.shape[1]
    return pl.pallas_call(
        _fc_body,
        grid=(1,),
        in_specs=[_cspec((n, 1, k)), _cspec((k, fp)), _cspec((1, fp))],
        out_specs=pl.BlockSpec((n, fp), lambda i: (0, 0)),
        out_shape=jax.ShapeDtypeStruct((n, fp), _F32),
    )(pooled, w, b.reshape(1, fp))


def kernel(x, f0_w, f0_b, f1_dw_w, f1_dw_b, f1_proj_w, f1_proj_b, f2_expand_w, f2_expand_b, f2_dw_w, f2_dw_b, f2_proj_w, f2_proj_b, f3_expand_w, f3_expand_b, f3_dw_w, f3_dw_b, f3_proj_w, f3_proj_b, f4_expand_w, f4_expand_b, f4_dw_w, f4_dw_b, f4_proj_w, f4_proj_b, f5_expand_w, f5_expand_b, f5_dw_w, f5_dw_b, f5_proj_w, f5_proj_b, f6_expand_w, f6_expand_b, f6_dw_w, f6_dw_b, f6_proj_w, f6_proj_b, f7_expand_w, f7_expand_b, f7_dw_w, f7_dw_b, f7_proj_w, f7_proj_b, f8_expand_w, f8_expand_b, f8_dw_w, f8_dw_b, f8_proj_w, f8_proj_b, f9_expand_w, f9_expand_b, f9_dw_w, f9_dw_b, f9_proj_w, f9_proj_b, f10_expand_w, f10_expand_b, f10_dw_w, f10_dw_b, f10_proj_w, f10_proj_b, f11_expand_w, f11_expand_b, f11_dw_w, f11_dw_b, f11_proj_w, f11_proj_b, f12_expand_w, f12_expand_b, f12_dw_w, f12_dw_b, f12_proj_w, f12_proj_b, f13_expand_w, f13_expand_b, f13_dw_w, f13_dw_b, f13_proj_w, f13_proj_b, f14_expand_w, f14_expand_b, f14_dw_w, f14_dw_b, f14_proj_w, f14_proj_b, f15_expand_w, f15_expand_b, f15_dw_w, f15_dw_b, f15_proj_w, f15_proj_b, f16_expand_w, f16_expand_b, f16_dw_w, f16_dw_b, f16_proj_w, f16_proj_b, f17_expand_w, f17_expand_b, f17_dw_w, f17_dw_b, f17_proj_w, f17_proj_b, f18_w, f18_b, fc_w, fc_b):
    n = x.shape[0]
    xbf = x.astype(_BF16)
    xi = jax.lax.bitcast_convert_type(
        xbf.reshape(n, 3, 224, 112, 2), jnp.int32)
    y1 = _stem_f1(xi, f0_w, f0_b, f1_dw_w, f1_dw_b, f1_proj_w, f1_proj_b)
    o17, pooled = _tail_chain(y1, [
        (f2_expand_w, f2_expand_b, f2_dw_w, f2_dw_b, f2_proj_w, f2_proj_b, 2, False),
        (f3_expand_w, f3_expand_b, f3_dw_w, f3_dw_b, f3_proj_w, f3_proj_b, 1, True),
        (f4_expand_w, f4_expand_b, f4_dw_w, f4_dw_b, f4_proj_w, f4_proj_b, 2, False),
        (f5_expand_w, f5_expand_b, f5_dw_w, f5_dw_b, f5_proj_w, f5_proj_b, 1, True),
        (f6_expand_w, f6_expand_b, f6_dw_w, f6_dw_b, f6_proj_w, f6_proj_b, 1, True),
        (f7_expand_w, f7_expand_b, f7_dw_w, f7_dw_b, f7_proj_w, f7_proj_b, 2, False),
        (f8_expand_w, f8_expand_b, f8_dw_w, f8_dw_b, f8_proj_w, f8_proj_b, 1, True),
        (f9_expand_w, f9_expand_b, f9_dw_w, f9_dw_b, f9_proj_w, f9_proj_b, 1, True),
        (f10_expand_w, f10_expand_b, f10_dw_w, f10_dw_b, f10_proj_w, f10_proj_b, 1, True),
        (f11_expand_w, f11_expand_b, f11_dw_w, f11_dw_b, f11_proj_w, f11_proj_b, 1, False),
        (f12_expand_w, f12_expand_b, f12_dw_w, f12_dw_b, f12_proj_w, f12_proj_b, 1, True),
        (f13_expand_w, f13_expand_b, f13_dw_w, f13_dw_b, f13_proj_w, f13_proj_b, 1, True),
        (f14_expand_w, f14_expand_b, f14_dw_w, f14_dw_b, f14_proj_w, f14_proj_b, 2, False),
        (f15_expand_w, f15_expand_b, f15_dw_w, f15_dw_b, f15_proj_w, f15_proj_b, 1, True),
        (f16_expand_w, f16_expand_b, f16_dw_w, f16_dw_b, f16_proj_w, f16_proj_b, 1, True),
        (f17_expand_w, f17_expand_b, f17_dw_w, f17_dw_b, f17_proj_w, f17_proj_b, 1, False),
    ], f18_w, f18_b)

    predict = _fc(pooled, fc_w, fc_b)[:, :12]

    feat = jnp.transpose(o17[..., :320].astype(_F32), (0, 3, 1, 2))
    feature = jnp.stack([feat[:n // 2], feat[n // 2:]])
    return feature, predict
